# baseline (speedup 1.0000x reference)
import functools

import jax
import jax.numpy as jnp
from jax import lax
from jax.experimental import pallas as pl
from jax.experimental.pallas import tpu as pltpu
from jax.experimental.pallas import tpu_sc as plsc

HIDDEN = 2048
EPS = 1e-12
L = 16
NSL = HIDDEN // L
C = 16


def _lane_allsum(v):
    lanes = lax.iota(jnp.int32, L)
    dnums = lax.GatherDimensionNumbers(
        offset_dims=(), collapsed_slice_dims=(0,), start_index_map=(0,)
    )
    for sh in (8, 4, 2, 1):
        perm = lanes ^ sh
        v = v + lax.gather(
            v, perm[:, None], dnums, (1,),
            mode=lax.GatherScatterMode.PROMISE_IN_BOUNDS,
        )
    return v


def _rsqrt_newton(v):
    xi = lax.bitcast_convert_type(v, jnp.int32)
    yi = jnp.int32(0x5F3759DF) - lax.shift_right_logical(xi, 1)
    y = lax.bitcast_convert_type(yi, jnp.float32)
    half_v = 0.5 * v
    for _ in range(3):
        y = y * (1.5 - half_v * y * y)
    return y


def _make_sc_call(n_tokens):
    info = plsc.get_sparse_core_info()
    nc, ns = info.num_cores, info.num_subcores
    nw = nc * ns
    tpw = n_tokens // nw
    nchunks = tpw // C

    mesh = plsc.VectorSubcoreMesh(core_axis_name="c", subcore_axis_name="s")

    @functools.partial(
        pl.kernel,
        out_type=jax.ShapeDtypeStruct((n_tokens, HIDDEN), jnp.float32),
        mesh=mesh,
        scratch_types=[
            pltpu.VMEM((tpw,), jnp.int32),
            pltpu.VMEM((C, HIDDEN), jnp.float32),
            pltpu.VMEM((HIDDEN,), jnp.float32),
            pltpu.VMEM((HIDDEN,), jnp.float32),
            pltpu.SemaphoreType.DMA,
        ],
    )
    def sc_call(ids_hbm, table_hbm, gamma_hbm, beta_hbm, out_hbm,
                idx_v, rows_v, gamma_v, beta_v, sem):
        wid = lax.axis_index("s") * nc + lax.axis_index("c")
        base = wid * tpw
        pltpu.sync_copy(ids_hbm.at[pl.ds(base, tpw)], idx_v)
        pltpu.sync_copy(gamma_hbm, gamma_v)
        pltpu.sync_copy(beta_hbm, beta_v)

        def chunk_body(g, _):
            pltpu.async_copy(
                table_hbm.at[idx_v.at[pl.ds(g * C, C)]], rows_v, sem
            ).wait()

            def tok_body(t, _):
                def acc_body(j, carry):
                    s, q = carry
                    x = rows_v[t, pl.ds(j * L, L)]
                    return s + x, q + x * x

                zeros = jnp.zeros((L,), jnp.float32)
                s, q = lax.fori_loop(0, NSL, acc_body, (zeros, zeros))
                mean_v = _lane_allsum(s) * (1.0 / HIDDEN)
                ex2_v = _lane_allsum(q) * (1.0 / HIDDEN)
                var_v = ex2_v - mean_v * mean_v
                scale_v = _rsqrt_newton(var_v + EPS)

                def norm_body(j, _):
                    sl = pl.ds(j * L, L)
                    x = rows_v[t, sl]
                    y = (x - mean_v) * scale_v
                    rows_v[t, sl] = y * gamma_v[sl] + beta_v[sl]
                    return 0

                lax.fori_loop(0, NSL, norm_body, 0)
                return 0

            lax.fori_loop(0, C, tok_body, 0)
            pltpu.sync_copy(rows_v, out_hbm.at[pl.ds(base + g * C, C)])
            return 0

        lax.fori_loop(0, nchunks, chunk_body, 0)

    return sc_call


def kernel(input_ids, word_embeddings, ln_gamma, ln_beta):
    b, s = input_ids.shape
    n = b * s
    ids = input_ids.reshape(n).astype(jnp.int32)
    out = _make_sc_call(n)(ids, word_embeddings, ln_gamma, ln_beta)
    return out.reshape(b, s, HIDDEN)
